# baseline (speedup 1.0000x reference)
import functools

import jax
import jax.numpy as jnp
from jax import lax
from jax.experimental import pallas as pl
from jax.experimental.pallas import tpu as pltpu
from jax.experimental.pallas import tpu_sc as plsc

NUM_TEAMS = 100000
NUM_FEATURES = 128
BATCH = 16384

_info = plsc.get_sparse_core_info()
_NC, _NS, _L = _info.num_cores, _info.num_subcores, _info.num_lanes
_NW = _NC * _NS
_BPW = BATCH // _NW


def _sc_gather_diff_body(strengths_hbm, home_hbm, away_hbm, out_hbm,
                         hidx_v, aidx_v, hval_v, aval_v, sem_h, sem_a):
    wid = lax.axis_index("s") * _NC + lax.axis_index("c")
    base = wid * _BPW
    cp_hi = pltpu.async_copy(home_hbm.at[pl.ds(base, _BPW)], hidx_v, sem_h)
    cp_ai = pltpu.async_copy(away_hbm.at[pl.ds(base, _BPW)], aidx_v, sem_a)
    cp_hi.wait()
    cp_h = pltpu.async_copy(strengths_hbm.at[hidx_v], hval_v, sem_h)
    cp_ai.wait()
    cp_a = pltpu.async_copy(strengths_hbm.at[aidx_v], aval_v, sem_a)
    cp_h.wait()
    cp_a.wait()
    for i in range(_BPW // _L):
        sl = pl.ds(i * _L, _L)
        hval_v[sl] = hval_v[sl] - aval_v[sl]
    pltpu.sync_copy(hval_v, out_hbm.at[pl.ds(base, _BPW)])


@jax.jit
def _sc_gather_diff(strengths, home_idx, away_idx):
    mesh = plsc.VectorSubcoreMesh(core_axis_name="c", subcore_axis_name="s")
    return pl.kernel(
        _sc_gather_diff_body,
        out_type=jax.ShapeDtypeStruct((BATCH,), jnp.float32),
        mesh=mesh,
        scratch_types=[
            pltpu.VMEM((_BPW,), jnp.int32),
            pltpu.VMEM((_BPW,), jnp.int32),
            pltpu.VMEM((_BPW,), jnp.float32),
            pltpu.VMEM((_BPW,), jnp.float32),
            pltpu.SemaphoreType.DMA,
            pltpu.SemaphoreType.DMA,
        ],
    )(strengths, home_idx, away_idx)


def _tc_matvec_body(x1_ref, x2_ref, beta_ref, mu_ref, out1_ref, out2_ref):
    b = beta_ref[...]
    s1 = jax.lax.dot_general(b, x1_ref[0], (((1,), (1,)), ((), ())),
                             preferred_element_type=jnp.float32)
    s2 = jax.lax.dot_general(b, x2_ref[0], (((1,), (1,)), ((), ())),
                             preferred_element_type=jnp.float32)
    out1_ref[...] = (s1 + mu_ref[0])[None]
    out2_ref[...] = (s2 + mu_ref[0])[None]


def _tc_add_body(y1_ref, y2_ref, diff_ref, out_ref):
    h = BATCH // 2
    out_ref[pl.ds(0, h)] = y1_ref[...] + diff_ref[pl.ds(0, h)]
    out_ref[pl.ds(h, h)] = y2_ref[...] + diff_ref[pl.ds(h, h)]


@jax.jit
def _fused(X, beta, mu, home_idx, away_idx, strengths):
    n_blocks = 4
    half = BATCH // 2
    bs = half // n_blocks
    X2 = X.reshape(2, half, NUM_FEATURES)
    y1, y2 = pl.pallas_call(
        _tc_matvec_body,
        grid=(n_blocks,),
        in_specs=[
            pl.BlockSpec((1, bs, NUM_FEATURES), lambda i: (0, i, 0)),
            pl.BlockSpec((1, bs, NUM_FEATURES), lambda i: (1, i, 0)),
            pl.BlockSpec((1, NUM_FEATURES), lambda i: (0, 0)),
            pl.BlockSpec((1,), lambda i: (0,)),
        ],
        out_specs=[
            pl.BlockSpec((1, 1, bs), lambda i: (i, 0, 0)),
            pl.BlockSpec((1, 1, bs), lambda i: (i, 0, 0)),
        ],
        out_shape=[
            jax.ShapeDtypeStruct((n_blocks, 1, bs), jnp.float32),
            jax.ShapeDtypeStruct((n_blocks, 1, bs), jnp.float32),
        ],
    )(X2, X2, beta.reshape(1, NUM_FEATURES), mu)
    diff = _sc_gather_diff(strengths, home_idx, away_idx)
    return pl.pallas_call(
        _tc_add_body,
        out_shape=jax.ShapeDtypeStruct((BATCH,), jnp.float32),
    )(y1.reshape(half), y2.reshape(half), diff)


def kernel(home_idx, away_idx, X, strengths, beta, mu):
    home_idx = home_idx.astype(jnp.int32)
    away_idx = away_idx.astype(jnp.int32)
    return _fused(X, beta, mu, home_idx, away_idx, strengths)


# speedup vs baseline: 1.0231x; 1.0231x over previous
import functools

import jax
import jax.numpy as jnp
from jax import lax
from jax.experimental import pallas as pl
from jax.experimental.pallas import tpu as pltpu
from jax.experimental.pallas import tpu_sc as plsc

NUM_TEAMS = 100000
NUM_FEATURES = 128
BATCH = 16384

_info = plsc.get_sparse_core_info()
_NC, _NS, _L = _info.num_cores, _info.num_subcores, _info.num_lanes
_NW = _NC * _NS
_BPW = BATCH // _NW


def _sc_gather_diff_body(strengths_hbm, home_hbm, away_hbm, out_hbm,
                         hidx_v, aidx_v, hval_v, aval_v, tbl_spmem,
                         sem_h, sem_a):
    wid = lax.axis_index("s") * _NC + lax.axis_index("c")
    base = wid * _BPW
    cp_hi = pltpu.async_copy(home_hbm.at[pl.ds(base, _BPW)], hidx_v, sem_h)
    cp_ai = pltpu.async_copy(away_hbm.at[pl.ds(base, _BPW)], aidx_v, sem_a)

    @pl.when(lax.axis_index("s") == 0)
    def _stage():
        pltpu.sync_copy(strengths_hbm, tbl_spmem)

    plsc.subcore_barrier()
    cp_hi.wait()
    cp_h = pltpu.async_copy(tbl_spmem.at[hidx_v], hval_v, sem_h)
    cp_ai.wait()
    cp_a = pltpu.async_copy(tbl_spmem.at[aidx_v], aval_v, sem_a)
    cp_h.wait()
    cp_a.wait()
    for i in range(_BPW // _L):
        sl = pl.ds(i * _L, _L)
        hval_v[sl] = hval_v[sl] - aval_v[sl]
    pltpu.sync_copy(hval_v, out_hbm.at[pl.ds(base, _BPW)])


@jax.jit
def _sc_gather_diff(strengths, home_idx, away_idx):
    mesh = plsc.VectorSubcoreMesh(core_axis_name="c", subcore_axis_name="s")
    return pl.kernel(
        _sc_gather_diff_body,
        out_type=jax.ShapeDtypeStruct((BATCH,), jnp.float32),
        mesh=mesh,
        scratch_types=[
            pltpu.VMEM((_BPW,), jnp.int32),
            pltpu.VMEM((_BPW,), jnp.int32),
            pltpu.VMEM((_BPW,), jnp.float32),
            pltpu.VMEM((_BPW,), jnp.float32),
            pltpu.VMEM_SHARED((NUM_TEAMS,), jnp.float32),
            pltpu.SemaphoreType.DMA,
            pltpu.SemaphoreType.DMA,
        ],
    )(strengths, home_idx, away_idx)


def _tc_matvec_body(x1_ref, x2_ref, beta_ref, mu_ref, out1_ref, out2_ref):
    b = beta_ref[...]
    s1 = jax.lax.dot_general(b, x1_ref[0], (((1,), (1,)), ((), ())),
                             preferred_element_type=jnp.float32)
    s2 = jax.lax.dot_general(b, x2_ref[0], (((1,), (1,)), ((), ())),
                             preferred_element_type=jnp.float32)
    out1_ref[...] = (s1 + mu_ref[0])[None]
    out2_ref[...] = (s2 + mu_ref[0])[None]


def _tc_add_body(y1_ref, y2_ref, diff_ref, out_ref):
    h = BATCH // 2
    out_ref[pl.ds(0, h)] = y1_ref[...] + diff_ref[pl.ds(0, h)]
    out_ref[pl.ds(h, h)] = y2_ref[...] + diff_ref[pl.ds(h, h)]


@jax.jit
def _fused(X, beta, mu, home_idx, away_idx, strengths):
    n_blocks = 4
    half = BATCH // 2
    bs = half // n_blocks
    X2 = X.reshape(2, half, NUM_FEATURES)
    y1, y2 = pl.pallas_call(
        _tc_matvec_body,
        grid=(n_blocks,),
        in_specs=[
            pl.BlockSpec((1, bs, NUM_FEATURES), lambda i: (0, i, 0)),
            pl.BlockSpec((1, bs, NUM_FEATURES), lambda i: (1, i, 0)),
            pl.BlockSpec((1, NUM_FEATURES), lambda i: (0, 0)),
            pl.BlockSpec((1,), lambda i: (0,)),
        ],
        out_specs=[
            pl.BlockSpec((1, 1, bs), lambda i: (i, 0, 0)),
            pl.BlockSpec((1, 1, bs), lambda i: (i, 0, 0)),
        ],
        out_shape=[
            jax.ShapeDtypeStruct((n_blocks, 1, bs), jnp.float32),
            jax.ShapeDtypeStruct((n_blocks, 1, bs), jnp.float32),
        ],
    )(X2, X2, beta.reshape(1, NUM_FEATURES), mu)
    diff = _sc_gather_diff(strengths, home_idx, away_idx)
    return pl.pallas_call(
        _tc_add_body,
        out_shape=jax.ShapeDtypeStruct((BATCH,), jnp.float32),
    )(y1.reshape(half), y2.reshape(half), diff)


def kernel(home_idx, away_idx, X, strengths, beta, mu):
    home_idx = home_idx.astype(jnp.int32)
    away_idx = away_idx.astype(jnp.int32)
    return _fused(X, beta, mu, home_idx, away_idx, strengths)


# speedup vs baseline: 1.0267x; 1.0035x over previous
import functools

import jax
import jax.numpy as jnp
from jax import lax
from jax.experimental import pallas as pl
from jax.experimental.pallas import tpu as pltpu
from jax.experimental.pallas import tpu_sc as plsc

NUM_TEAMS = 100000
NUM_FEATURES = 128
BATCH = 16384

_info = plsc.get_sparse_core_info()
_NC, _NS, _L = _info.num_cores, _info.num_subcores, _info.num_lanes
_NW = _NC * _NS
_BPW = BATCH // _NW


def _sc_gather_diff_body(strengths_hbm, home_hbm, away_hbm, out_hbm,
                         hidx_v, aidx_v, hval_v, aval_v, tbl_spmem,
                         sem_h, sem_a):
    wid = lax.axis_index("s") * _NC + lax.axis_index("c")
    base = wid * _BPW
    cp_hi = pltpu.async_copy(home_hbm.at[pl.ds(base, _BPW)], hidx_v, sem_h)
    cp_ai = pltpu.async_copy(away_hbm.at[pl.ds(base, _BPW)], aidx_v, sem_a)

    @pl.when(lax.axis_index("s") == 0)
    def _stage():
        pltpu.sync_copy(strengths_hbm, tbl_spmem)

    plsc.subcore_barrier()
    cp_hi.wait()
    cp_h = pltpu.async_copy(tbl_spmem.at[hidx_v], hval_v, sem_h)
    cp_ai.wait()
    cp_a = pltpu.async_copy(tbl_spmem.at[aidx_v], aval_v, sem_a)
    cp_h.wait()
    cp_a.wait()
    for i in range(_BPW // _L):
        sl = pl.ds(i * _L, _L)
        hval_v[sl] = hval_v[sl] - aval_v[sl]
    pltpu.sync_copy(hval_v, out_hbm.at[pl.ds(base, _BPW)])


@jax.jit
def _sc_gather_diff(strengths, home_idx, away_idx):
    mesh = plsc.VectorSubcoreMesh(core_axis_name="c", subcore_axis_name="s")
    return pl.kernel(
        _sc_gather_diff_body,
        out_type=jax.ShapeDtypeStruct((BATCH,), jnp.float32),
        mesh=mesh,
        scratch_types=[
            pltpu.VMEM((_BPW,), jnp.int32),
            pltpu.VMEM((_BPW,), jnp.int32),
            pltpu.VMEM((_BPW,), jnp.float32),
            pltpu.VMEM((_BPW,), jnp.float32),
            pltpu.VMEM_SHARED((NUM_TEAMS,), jnp.float32),
            pltpu.SemaphoreType.DMA,
            pltpu.SemaphoreType.DMA,
        ],
    )(strengths, home_idx, away_idx)


def _tc_matvec_body(x1_ref, x2_ref, x3_ref, x4_ref, beta_ref, mu_ref,
                    out1_ref, out2_ref, out3_ref, out4_ref):
    b = beta_ref[...]
    mu = mu_ref[0]
    for x_ref, o_ref in ((x1_ref, out1_ref), (x2_ref, out2_ref),
                         (x3_ref, out3_ref), (x4_ref, out4_ref)):
        s = jax.lax.dot_general(b, x_ref[0], (((1,), (1,)), ((), ())),
                                preferred_element_type=jnp.float32)
        o_ref[...] = (s + mu)[None]


def _tc_add_body(y1_ref, y2_ref, y3_ref, y4_ref, diff_ref, out_ref):
    q = BATCH // 4
    for k, y_ref in enumerate((y1_ref, y2_ref, y3_ref, y4_ref)):
        out_ref[pl.ds(k * q, q)] = y_ref[...] + diff_ref[pl.ds(k * q, q)]


@jax.jit
def _fused(X, beta, mu, home_idx, away_idx, strengths):
    n_blocks = 2
    quarter = BATCH // 4
    bs = quarter // n_blocks
    X4 = X.reshape(4, quarter, NUM_FEATURES)
    x_spec = [
        pl.BlockSpec((1, bs, NUM_FEATURES), lambda i, k=k: (k, i, 0))
        for k in range(4)
    ]
    o_spec = [pl.BlockSpec((1, 1, bs), lambda i: (i, 0, 0)) for _ in range(4)]
    ys = pl.pallas_call(
        _tc_matvec_body,
        grid=(n_blocks,),
        in_specs=x_spec + [
            pl.BlockSpec((1, NUM_FEATURES), lambda i: (0, 0)),
            pl.BlockSpec((1,), lambda i: (0,)),
        ],
        out_specs=o_spec,
        out_shape=[jax.ShapeDtypeStruct((n_blocks, 1, bs), jnp.float32)
                   for _ in range(4)],
    )(X4, X4, X4, X4, beta.reshape(1, NUM_FEATURES), mu)
    diff = _sc_gather_diff(strengths, home_idx, away_idx)
    return pl.pallas_call(
        _tc_add_body,
        out_shape=jax.ShapeDtypeStruct((BATCH,), jnp.float32),
    )(*[y.reshape(quarter) for y in ys], diff)


def kernel(home_idx, away_idx, X, strengths, beta, mu):
    home_idx = home_idx.astype(jnp.int32)
    away_idx = away_idx.astype(jnp.int32)
    return _fused(X, beta, mu, home_idx, away_idx, strengths)
